# baseline (speedup 1.0000x reference)
import functools

import jax
import jax.numpy as jnp
from jax import lax
from jax.experimental import pallas as pl
from jax.experimental.pallas import tpu as pltpu
from jax.experimental.pallas import tpu_sc as plsc

_B, _C, _P, _S, _K = 8, 128, 4096, 1024, 16
_SBLK = 256


def _topk_body(xx_ref, xx2_ref, x_ref, x2_ref, idx_ref):
    xb = x_ref[0]
    x2b = x2_ref[0]
    dot = lax.dot_general(
        x2b, xb, dimension_numbers=(((0,), (0,)), ((), ())),
        precision=lax.Precision.DEFAULT,
        preferred_element_type=jnp.float32)
    inner = -2.0 * dot
    pd = (-xx_ref[0, 0][None, :] - inner) - xx2_ref[0, 0][:, None]
    iotaf = lax.broadcasted_iota(jnp.int32, pd.shape, 1).astype(jnp.float32)
    neg_inf = jnp.float32(-jnp.inf)
    for j in range(_K):
        m = jnp.max(pd, axis=1, keepdims=True)
        af = jnp.min(jnp.where(pd == m, iotaf, jnp.float32(_P)), axis=1)
        idx_ref[0, j, :] = af.astype(jnp.int32)
        if j + 1 < _K:
            pd = jnp.where(iotaf == af[:, None], neg_inf, pd)


def _topk_indices(x, x2, xx, xx2):
    return pl.pallas_call(
        _topk_body,
        grid=(_B, _S // _SBLK),
        in_specs=[
            pl.BlockSpec((1, 1, _P), lambda b, j: (b, 0, 0)),
            pl.BlockSpec((1, 1, _SBLK), lambda b, j: (b, 0, j)),
            pl.BlockSpec((1, _C, _P), lambda b, j: (b, 0, 0)),
            pl.BlockSpec((1, _C, _SBLK), lambda b, j: (b, 0, j)),
        ],
        out_specs=pl.BlockSpec((1, _K, _SBLK), lambda b, j: (b, 0, j)),
        out_shape=jax.ShapeDtypeStruct((_B, _K, _S), jnp.int32),
    )(xx.reshape(_B, 1, _P), xx2.reshape(_B, 1, _S), x, x2)


_ROWS_PER_W = (_B * _C) // 32
_T = _K * _S


_TCH = 4


def _gather_body(xf_hbm, idx_hbm, out_hbm,
                 idx_v, t_a, t_b, r_0, r_1, st_a, st_b, so_0, so_1):
    wid = lax.axis_index("c") * 16 + lax.axis_index("s")
    b = wid // 4
    c0 = (wid % 4) * _ROWS_PER_W
    base = b * _C + c0
    pltpu.sync_copy(idx_hbm.at[b], idx_v)
    pltpu.async_copy(xf_hbm.at[pl.ds(base, _TCH)], t_a, st_a)
    pltpu.async_copy(out_hbm.at[b, c0], r_0, so_0)
    pltpu.async_copy(out_hbm.at[b, c0], r_1, so_1)
    rbufs = ((r_0, so_0), (r_1, so_1))

    def do_row(tbuf, r, rbuf, rsem, j):
        cvec = jnp.full((16,), r % _TCH, jnp.int32)
        pltpu.make_async_copy(rbuf, out_hbm.at[b, 0], rsem).wait()

        @plsc.parallel_loop(0, _S // 16, unroll=2)
        def _(s0, cvec=cvec, rbuf=rbuf, tbuf=tbuf):
            for kk in range(_K):
                iv = idx_v[pl.ds(kk * _S + s0 * 16, 16)]
                rbuf[kk, pl.ds(s0 * 16, 16)] = plsc.load_gather(
                    tbuf, [cvec, iv])

        pltpu.async_copy(rbuf, out_hbm.at[b, c0 + j + r], rsem)

    @pl.loop(0, _ROWS_PER_W, step=2 * _TCH)
    def _(j):
        pltpu.make_async_copy(xf_hbm.at[pl.ds(base, _TCH)], t_a, st_a).wait()
        pltpu.async_copy(xf_hbm.at[pl.ds(base + j + _TCH, _TCH)], t_b, st_b)
        for r in range(_TCH):
            do_row(t_a, r, *rbufs[r % 2], j)
        pltpu.make_async_copy(xf_hbm.at[pl.ds(base, _TCH)], t_b, st_b).wait()
        nxt = jnp.minimum(j + 2 * _TCH, _ROWS_PER_W - _TCH)
        pltpu.async_copy(xf_hbm.at[pl.ds(base + nxt, _TCH)], t_a, st_a)
        for r in range(_TCH, 2 * _TCH):
            do_row(t_b, r, *rbufs[r % 2], j)

    pltpu.make_async_copy(xf_hbm.at[pl.ds(base, _TCH)], t_a, st_a).wait()
    for rbuf, rsem in rbufs:
        pltpu.make_async_copy(rbuf, out_hbm.at[b, 0], rsem).wait()


def _gather(xf, idxflat):
    mesh = plsc.VectorSubcoreMesh(core_axis_name="c", subcore_axis_name="s")
    run = functools.partial(
        pl.kernel,
        out_type=jax.ShapeDtypeStruct((_B, _C, _K, _S), jnp.float32),
        mesh=mesh,
        scratch_types=[
            pltpu.VMEM((_T,), jnp.int32),
            pltpu.VMEM((_TCH, _P), jnp.float32),
            pltpu.VMEM((_TCH, _P), jnp.float32),
            pltpu.VMEM((_K, _S), jnp.float32),
            pltpu.VMEM((_K, _S), jnp.float32),
            pltpu.SemaphoreType.DMA,
            pltpu.SemaphoreType.DMA,
            pltpu.SemaphoreType.DMA,
            pltpu.SemaphoreType.DMA,
        ],
        compiler_params=pltpu.CompilerParams(needs_layout_passes=False),
    )(_gather_body)
    return run(xf, idxflat)


def kernel(x, x2, k):
    B, C, P = x.shape
    S = x2.shape[2]
    xx = jnp.sum(x * x, axis=1)
    xx2 = jnp.sum(x2 * x2, axis=1)
    idx = _topk_indices(x, x2, xx, xx2)
    idx = idx + (jnp.asarray(k) - _K).astype(idx.dtype)
    idxp = jnp.transpose(idx.reshape(B, S, _K), (0, 2, 1)).reshape(B, _T)
    xf = x.reshape(B * C, P)
    out4 = _gather(xf, idxp)
    return jnp.transpose(out4, (0, 1, 3, 2))


# speedup vs baseline: 1.0049x; 1.0049x over previous
import functools

import jax
import jax.numpy as jnp
from jax import lax
from jax.experimental import pallas as pl
from jax.experimental.pallas import tpu as pltpu
from jax.experimental.pallas import tpu_sc as plsc

_B, _C, _P, _S, _K = 8, 128, 4096, 1024, 16
_SBLK = 256


def _topk_body(xx_ref, xx2_ref, x_ref, x2_ref, idx_ref):
    xb = x_ref[0]
    x2b = x2_ref[0]
    dot = lax.dot_general(
        x2b, xb, dimension_numbers=(((0,), (0,)), ((), ())),
        precision=lax.Precision.DEFAULT,
        preferred_element_type=jnp.float32)
    inner = -2.0 * dot
    pd = (-xx_ref[0, 0][None, :] - inner) - xx2_ref[0, 0][:, None]
    iotaf = lax.broadcasted_iota(jnp.int32, pd.shape, 1).astype(jnp.float32)
    neg_inf = jnp.float32(-jnp.inf)
    for j in range(_K):
        m = jnp.max(pd, axis=1, keepdims=True)
        af = jnp.min(jnp.where(pd == m, iotaf, jnp.float32(_P)), axis=1)
        idx_ref[0, j, :] = af.astype(jnp.int32)
        if j + 1 < _K:
            pd = jnp.where(iotaf == af[:, None], neg_inf, pd)


def _topk_indices(x, x2, xx, xx2):
    return pl.pallas_call(
        _topk_body,
        grid=(_B, _S // _SBLK),
        in_specs=[
            pl.BlockSpec((1, 1, _P), lambda b, j: (b, 0, 0)),
            pl.BlockSpec((1, 1, _SBLK), lambda b, j: (b, 0, j)),
            pl.BlockSpec((1, _C, _P), lambda b, j: (b, 0, 0)),
            pl.BlockSpec((1, _C, _SBLK), lambda b, j: (b, 0, j)),
        ],
        out_specs=pl.BlockSpec((1, _K, _SBLK), lambda b, j: (b, 0, j)),
        out_shape=jax.ShapeDtypeStruct((_B, _K, _S), jnp.int32),
    )(xx.reshape(_B, 1, _P), xx2.reshape(_B, 1, _S), x, x2)


_ROWS_PER_W = (_B * _C) // 32
_T = _K * _S


_TCH = 4


def _gather_body(xf_hbm, idx_hbm, out_hbm,
                 idx_v, t_a, t_b, r_0, r_1, st_a, st_b, so_0, so_1):
    wid = lax.axis_index("c") * 16 + lax.axis_index("s")
    b = wid // 4
    c0 = (wid % 4) * _ROWS_PER_W
    base = b * _C + c0
    pltpu.sync_copy(idx_hbm.at[b], idx_v)
    pltpu.async_copy(xf_hbm.at[pl.ds(base, _TCH)], t_a, st_a)
    pltpu.async_copy(out_hbm.at[b, c0], r_0, so_0)
    pltpu.async_copy(out_hbm.at[b, c0], r_1, so_1)
    rbufs = ((r_0, so_0), (r_1, so_1))

    def do_row(tbuf, r, rbuf, rsem, j):
        cvec = jnp.full((16,), r % _TCH, jnp.int32)
        pltpu.make_async_copy(rbuf, out_hbm.at[b, 0], rsem).wait()

        @plsc.parallel_loop(0, _S // 16)
        def _(s0, cvec=cvec, rbuf=rbuf, tbuf=tbuf):
            for kk in range(_K):
                iv = idx_v[pl.ds(kk * _S + s0 * 16, 16)]
                rbuf[kk, pl.ds(s0 * 16, 16)] = plsc.load_gather(
                    tbuf, [cvec, iv])

        pltpu.async_copy(rbuf, out_hbm.at[b, c0 + j + r], rsem)

    @pl.loop(0, _ROWS_PER_W, step=2 * _TCH)
    def _(j):
        pltpu.make_async_copy(xf_hbm.at[pl.ds(base, _TCH)], t_a, st_a).wait()
        pltpu.async_copy(xf_hbm.at[pl.ds(base + j + _TCH, _TCH)], t_b, st_b)
        for r in range(_TCH):
            do_row(t_a, r, *rbufs[r % 2], j)
        pltpu.make_async_copy(xf_hbm.at[pl.ds(base, _TCH)], t_b, st_b).wait()
        nxt = jnp.minimum(j + 2 * _TCH, _ROWS_PER_W - _TCH)
        pltpu.async_copy(xf_hbm.at[pl.ds(base + nxt, _TCH)], t_a, st_a)
        for r in range(_TCH, 2 * _TCH):
            do_row(t_b, r, *rbufs[r % 2], j)

    pltpu.make_async_copy(xf_hbm.at[pl.ds(base, _TCH)], t_a, st_a).wait()
    for rbuf, rsem in rbufs:
        pltpu.make_async_copy(rbuf, out_hbm.at[b, 0], rsem).wait()


def _gather(xf, idxflat):
    mesh = plsc.VectorSubcoreMesh(core_axis_name="c", subcore_axis_name="s")
    run = functools.partial(
        pl.kernel,
        out_type=jax.ShapeDtypeStruct((_B, _C, _K, _S), jnp.float32),
        mesh=mesh,
        scratch_types=[
            pltpu.VMEM((_T,), jnp.int32),
            pltpu.VMEM((_TCH, _P), jnp.float32),
            pltpu.VMEM((_TCH, _P), jnp.float32),
            pltpu.VMEM((_K, _S), jnp.float32),
            pltpu.VMEM((_K, _S), jnp.float32),
            pltpu.SemaphoreType.DMA,
            pltpu.SemaphoreType.DMA,
            pltpu.SemaphoreType.DMA,
            pltpu.SemaphoreType.DMA,
        ],
        compiler_params=pltpu.CompilerParams(needs_layout_passes=False),
    )(_gather_body)
    return run(xf, idxflat)


def kernel(x, x2, k):
    B, C, P = x.shape
    S = x2.shape[2]
    xx = jnp.sum(x * x, axis=1)
    xx2 = jnp.sum(x2 * x2, axis=1)
    idx = _topk_indices(x, x2, xx, xx2)
    idx = idx + (jnp.asarray(k) - _K).astype(idx.dtype)
    idxp = jnp.transpose(idx.reshape(B, S, _K), (0, 2, 1)).reshape(B, _T)
    xf = x.reshape(B * C, P)
    out4 = _gather(xf, idxp)
    return jnp.transpose(out4, (0, 1, 3, 2))
